# baseline (speedup 1.0000x reference)
import functools

import jax
import jax.numpy as jnp
from jax import lax
from jax.experimental import pallas as pl
from jax.experimental.pallas import tpu as pltpu
from jax.experimental.pallas import tpu_sc as plsc

K = 20
TN = 256
NC, NS = 2, 16
NW = NC * NS
CP = 32
IPG = 80


def _knn_body(xf_ref, xt_ref, w_ref, s_ref, idx_ref, us_ref, vt_ref):
    n = xf_ref.shape[2]
    c = xf_ref.shape[1]
    xf = xf_ref[0]
    xa = xt_ref[0]
    xtt = xa.T
    wt = w_ref[...].T
    w1t = wt[:c, :]
    wdt = wt[c:, :] - w1t

    inner = jnp.dot(xtt, xf, preferred_element_type=jnp.float32)
    xx_all = jnp.sum(xf * xf, axis=0, keepdims=True)
    xx_row = jnp.sum(xtt * xtt, axis=1, keepdims=True)
    d = 2.0 * inner - xx_row - xx_all

    b = pl.program_id(0)
    iot = lax.broadcasted_iota(jnp.int32, (TN, n), 1).astype(jnp.float32)
    nf = jnp.float32(n)
    neg = jnp.float32(-jnp.inf)
    cur = d
    for r in range(K):
        m = jnp.max(cur, axis=1, keepdims=True)
        cand = jnp.where(cur == m, iot, nf)
        col = jnp.min(cand, axis=1, keepdims=True)
        idx_ref[0, :, pl.ds(r, 1)] = col.astype(jnp.int32) + b * n
        cur = jnp.where(iot == col, neg, cur)

    us = jnp.dot(xtt, w1t, preferred_element_type=jnp.float32)
    usr = us * s_ref[...]
    us_ref[...] = jnp.concatenate([usr, usr * usr], axis=1)
    vt_ref[...] = jnp.dot(xtt, wdt, preferred_element_type=jnp.float32)


def _sc_gather_body(idx_hbm, us_hbm, ms_hbm, gs_hbm, g2_hbm,
                    idx_v, rows_v, ms_v, gs_v, g2_v, sem):
    wid = lax.axis_index("s") * NC + lax.axis_index("c")
    ppw = ms_hbm.shape[0] // NW
    nch = ppw // CP

    def chunk_body(ci, carry):
        pbase = pl.multiple_of(wid * ppw + ci * CP, 8)
        irow = pl.multiple_of((wid * ppw + ci * CP) * K // IPG, 8)
        pltpu.sync_copy(idx_hbm.at[pl.ds(irow, (CP * K) // IPG)], idx_v)
        nblk = (CP * K) // IPG
        ppb = IPG // K
        cps = []
        for i in range(nblk):
            cps.append(pltpu.async_copy(
                us_hbm.at[idx_v.at[i]],
                rows_v.at[pl.ds(i * IPG, IPG)], sem))

        for i in range(nblk):
            cps[i].wait()

            def p_body(p, c2, i=i):
                pp = i * ppb + p
                base = pp * K
                for j in range(4):
                    sl = pl.ds(j * 16, 16)
                    r0 = rows_v[base, sl]
                    m = r0
                    g = r0
                    for k in range(1, K):
                        rk = rows_v[base + k, sl]
                        m = jnp.maximum(m, rk)
                        g = g + rk
                    ms_v[pp, sl] = m
                    gs_v[pp, sl] = g
                for j in range(4):
                    sl = pl.ds(j * 16, 16)
                    slr = pl.ds(64 + j * 16, 16)
                    g2 = rows_v[base, slr]
                    for k in range(1, K):
                        g2 = g2 + rows_v[base + k, slr]
                    g2_v[pp, sl] = g2
                return c2

            lax.fori_loop(0, ppb, p_body, 0)
        pltpu.sync_copy(ms_v, ms_hbm.at[pl.ds(pbase, CP)])
        pltpu.sync_copy(gs_v, gs_hbm.at[pl.ds(pbase, CP)])
        pltpu.sync_copy(g2_v, g2_hbm.at[pl.ds(pbase, CP)])
        return carry

    lax.fori_loop(0, nch, chunk_body, 0)


def _finalize_body(*refs):
    part_refs = refs[:-4]
    s_ref, gam_ref, bet_ref, out_ref = refs[-4:]
    nparts = len(part_refs) // 4
    bsz, outc, n = out_ref.shape
    hb = bsz // nparts
    s = s_ref[...]
    kf = jnp.float32(K)
    cnt = jnp.float32(bsz * n * K)

    halves = []
    sum_y = 0.0
    sum_y2 = 0.0
    for ms_ref, gs_ref, g2_ref, vt_ref in (
            part_refs[4 * i:4 * i + 4] for i in range(nparts)):
        v = vt_ref[...]
        gs = gs_ref[...] * s
        e = ms_ref[...] * s + v
        sum_y = sum_y + jnp.sum(gs, axis=0, keepdims=True) + kf * jnp.sum(
            v, axis=0, keepdims=True)
        sum_y2 = (sum_y2 + jnp.sum(g2_ref[...], axis=0, keepdims=True)
                  + 2.0 * jnp.sum(gs * v, axis=0, keepdims=True)
                  + kf * jnp.sum(v * v, axis=0, keepdims=True))
        halves.append(e)

    mean = sum_y / cnt
    var = sum_y2 / cnt - mean * mean
    inv = 1.0 / jnp.sqrt(var + 1e-5)
    a = gam_ref[...] * inv
    bb = bet_ref[...] - mean * a
    for i, e in enumerate(halves):
        z = e * a + bb
        z = jnp.where(z > 0, z, 0.2 * z)
        out_ref[pl.ds(i * hb, hb)] = jnp.transpose(
            z.reshape(hb, n, outc), (0, 2, 1))


NSPLIT = 4


def kernel(x, W, gamma, beta):
    b, c, n = x.shape
    outc = W.shape[0]
    hb = b // NSPLIT
    hp = hb * n

    s = jnp.where(gamma >= 0, 1.0, -1.0).astype(jnp.float32).reshape(1, outc)

    knn = pl.pallas_call(
        _knn_body,
        grid=(hb, n // TN),
        in_specs=[
            pl.BlockSpec((1, c, n), lambda bi, ti: (bi, 0, 0)),
            pl.BlockSpec((1, c, TN), lambda bi, ti: (bi, 0, ti)),
            pl.BlockSpec((outc, 2 * c), lambda bi, ti: (0, 0)),
            pl.BlockSpec((1, outc), lambda bi, ti: (0, 0)),
        ],
        out_specs=[
            pl.BlockSpec((1, TN, K), lambda bi, ti: (bi, ti, 0)),
            pl.BlockSpec((TN, 2 * outc),
                         lambda bi, ti: (bi * (n // TN) + ti, 0)),
            pl.BlockSpec((TN, outc), lambda bi, ti: (bi * (n // TN) + ti, 0)),
        ],
        out_shape=[
            jax.ShapeDtypeStruct((hb, n, K), jnp.int32),
            jax.ShapeDtypeStruct((hp, 2 * outc), jnp.float32),
            jax.ShapeDtypeStruct((hp, outc), jnp.float32),
        ],
    )

    sc_gather = functools.partial(
        pl.kernel,
        mesh=plsc.VectorSubcoreMesh(core_axis_name="c", subcore_axis_name="s"),
        out_type=[jax.ShapeDtypeStruct((hp, outc), jnp.float32)] * 3,
        scratch_types=[
            pltpu.VMEM(((CP * K) // IPG, IPG), jnp.int32),
            pltpu.VMEM((CP * K, 2 * outc), jnp.float32),
            pltpu.VMEM((CP, outc), jnp.float32),
            pltpu.VMEM((CP, outc), jnp.float32),
            pltpu.VMEM((CP, outc), jnp.float32),
            pltpu.SemaphoreType.DMA,
        ],
    )(_sc_gather_body)

    parts = []
    for i in range(NSPLIT):
        xh = lax.slice_in_dim(x, i * hb, (i + 1) * hb, axis=0)
        idx, us, vt = knn(xh, xh, W, s)
        idx2 = idx.reshape((hp * K) // IPG, IPG)
        ms, gs, g2 = sc_gather(idx2, us)
        parts += [ms, gs, g2, vt]

    finalize = pl.pallas_call(
        _finalize_body,
        out_shape=jax.ShapeDtypeStruct((b, outc, n), jnp.float32),
    )
    return finalize(*parts, s,
                    gamma.astype(jnp.float32).reshape(1, outc),
                    beta.astype(jnp.float32).reshape(1, outc))


# speedup vs baseline: 1.0175x; 1.0175x over previous
import functools

import jax
import jax.numpy as jnp
from jax import lax
from jax.experimental import pallas as pl
from jax.experimental.pallas import tpu as pltpu
from jax.experimental.pallas import tpu_sc as plsc

K = 20
TN = 256
NC, NS = 2, 16
NW = NC * NS
CP = 32
IPG = 80


def _knn_body(xf_ref, xt_ref, w_ref, s_ref, idx_ref, us_ref, vt_ref):
    n = xf_ref.shape[2]
    c = xf_ref.shape[1]
    xf = xf_ref[0]
    xa = xt_ref[0]
    xtt = xa.T
    wt = w_ref[...].T
    w1t = wt[:c, :]
    wdt = wt[c:, :] - w1t

    inner = jnp.dot(xtt, xf, preferred_element_type=jnp.float32)
    xx_all = jnp.sum(xf * xf, axis=0, keepdims=True)
    xx_row = jnp.sum(xtt * xtt, axis=1, keepdims=True)
    d = 2.0 * inner - xx_row - xx_all

    b = pl.program_id(0)
    iot = lax.broadcasted_iota(jnp.int32, (TN, n), 1).astype(jnp.float32)
    nf = jnp.float32(n)
    neg = jnp.float32(-jnp.inf)
    cur = d
    for r in range(K):
        m = jnp.max(cur, axis=1, keepdims=True)
        cand = jnp.where(cur == m, iot, nf)
        col = jnp.min(cand, axis=1, keepdims=True)
        idx_ref[0, :, pl.ds(r, 1)] = col.astype(jnp.int32) + b * n
        cur = jnp.where(iot == col, neg, cur)

    us = jnp.dot(xtt, w1t, preferred_element_type=jnp.float32)
    usr = us * s_ref[...]
    us_ref[...] = jnp.concatenate([usr, usr * usr], axis=1)
    vt_ref[...] = jnp.dot(xtt, wdt, preferred_element_type=jnp.float32)


def _sc_gather_body(idx_hbm, us_hbm, ms_hbm, gs_hbm, g2_hbm,
                    idx_v, rows_v, ms_v, gs_v, g2_v, sem):
    wid = lax.axis_index("s") * NC + lax.axis_index("c")
    ppw = ms_hbm.shape[0] // NW
    nch = ppw // CP

    def chunk_body(ci, carry):
        pbase = pl.multiple_of(wid * ppw + ci * CP, 8)
        irow = pl.multiple_of((wid * ppw + ci * CP) * K // IPG, 8)
        pltpu.sync_copy(idx_hbm.at[pl.ds(irow, (CP * K) // IPG)], idx_v)
        nblk = (CP * K) // IPG
        ppb = IPG // K
        cps = []
        for i in range(nblk):
            cps.append(pltpu.async_copy(
                us_hbm.at[idx_v.at[i]],
                rows_v.at[pl.ds(i * IPG, IPG)], sem))

        for i in range(nblk):
            cps[i].wait()

            def p_body(p, c2, i=i):
                pp = i * ppb + p
                base = pp * K
                for j in range(4):
                    sl = pl.ds(j * 16, 16)
                    r0 = rows_v[base, sl]
                    m = r0
                    g = r0
                    for k in range(1, K):
                        rk = rows_v[base + k, sl]
                        m = jnp.maximum(m, rk)
                        g = g + rk
                    ms_v[pp, sl] = m
                    gs_v[pp, sl] = g
                for j in range(4):
                    sl = pl.ds(j * 16, 16)
                    slr = pl.ds(64 + j * 16, 16)
                    g2 = rows_v[base, slr]
                    for k in range(1, K):
                        g2 = g2 + rows_v[base + k, slr]
                    g2_v[pp, sl] = g2
                return c2

            lax.fori_loop(0, ppb, p_body, 0)
        pltpu.sync_copy(ms_v, ms_hbm.at[pl.ds(pbase, CP)])
        pltpu.sync_copy(gs_v, gs_hbm.at[pl.ds(pbase, CP)])
        pltpu.sync_copy(g2_v, g2_hbm.at[pl.ds(pbase, CP)])
        return carry

    lax.fori_loop(0, nch, chunk_body, 0)


def _finalize_body(*refs):
    part_refs = refs[:-4]
    s_ref, gam_ref, bet_ref, out_ref = refs[-4:]
    nparts = len(part_refs) // 4
    bsz, outc, n = out_ref.shape
    hb = bsz // nparts
    s = s_ref[...]
    kf = jnp.float32(K)
    cnt = jnp.float32(bsz * n * K)

    halves = []
    sum_y = 0.0
    sum_y2 = 0.0
    for ms_ref, gs_ref, g2_ref, vt_ref in (
            part_refs[4 * i:4 * i + 4] for i in range(nparts)):
        v = vt_ref[...]
        gs = gs_ref[...] * s
        e = ms_ref[...] * s + v
        sum_y = sum_y + jnp.sum(gs, axis=0, keepdims=True) + kf * jnp.sum(
            v, axis=0, keepdims=True)
        sum_y2 = (sum_y2 + jnp.sum(g2_ref[...], axis=0, keepdims=True)
                  + 2.0 * jnp.sum(gs * v, axis=0, keepdims=True)
                  + kf * jnp.sum(v * v, axis=0, keepdims=True))
        halves.append(e)

    mean = sum_y / cnt
    var = sum_y2 / cnt - mean * mean
    inv = 1.0 / jnp.sqrt(var + 1e-5)
    a = gam_ref[...] * inv
    bb = bet_ref[...] - mean * a
    for i, e in enumerate(halves):
        z = e * a + bb
        z = jnp.where(z > 0, z, 0.2 * z)
        out_ref[pl.ds(i * hb, hb)] = jnp.transpose(
            z.reshape(hb, n, outc), (0, 2, 1))


NSPLIT = 8


def kernel(x, W, gamma, beta):
    b, c, n = x.shape
    outc = W.shape[0]
    hb = b // NSPLIT
    hp = hb * n

    s = jnp.where(gamma >= 0, 1.0, -1.0).astype(jnp.float32).reshape(1, outc)

    knn = pl.pallas_call(
        _knn_body,
        grid=(hb, n // TN),
        in_specs=[
            pl.BlockSpec((1, c, n), lambda bi, ti: (bi, 0, 0)),
            pl.BlockSpec((1, c, TN), lambda bi, ti: (bi, 0, ti)),
            pl.BlockSpec((outc, 2 * c), lambda bi, ti: (0, 0)),
            pl.BlockSpec((1, outc), lambda bi, ti: (0, 0)),
        ],
        out_specs=[
            pl.BlockSpec((1, TN, K), lambda bi, ti: (bi, ti, 0)),
            pl.BlockSpec((TN, 2 * outc),
                         lambda bi, ti: (bi * (n // TN) + ti, 0)),
            pl.BlockSpec((TN, outc), lambda bi, ti: (bi * (n // TN) + ti, 0)),
        ],
        out_shape=[
            jax.ShapeDtypeStruct((hb, n, K), jnp.int32),
            jax.ShapeDtypeStruct((hp, 2 * outc), jnp.float32),
            jax.ShapeDtypeStruct((hp, outc), jnp.float32),
        ],
    )

    sc_gather = functools.partial(
        pl.kernel,
        mesh=plsc.VectorSubcoreMesh(core_axis_name="c", subcore_axis_name="s"),
        out_type=[jax.ShapeDtypeStruct((hp, outc), jnp.float32)] * 3,
        scratch_types=[
            pltpu.VMEM(((CP * K) // IPG, IPG), jnp.int32),
            pltpu.VMEM((CP * K, 2 * outc), jnp.float32),
            pltpu.VMEM((CP, outc), jnp.float32),
            pltpu.VMEM((CP, outc), jnp.float32),
            pltpu.VMEM((CP, outc), jnp.float32),
            pltpu.SemaphoreType.DMA,
        ],
    )(_sc_gather_body)

    parts = []
    for i in range(NSPLIT):
        xh = lax.slice_in_dim(x, i * hb, (i + 1) * hb, axis=0)
        idx, us, vt = knn(xh, xh, W, s)
        idx2 = idx.reshape((hp * K) // IPG, IPG)
        ms, gs, g2 = sc_gather(idx2, us)
        parts += [ms, gs, g2, vt]

    finalize = pl.pallas_call(
        _finalize_body,
        out_shape=jax.ShapeDtypeStruct((b, outc, n), jnp.float32),
    )
    return finalize(*parts, s,
                    gamma.astype(jnp.float32).reshape(1, outc),
                    beta.astype(jnp.float32).reshape(1, outc))
